# baseline (speedup 1.0000x reference)
import jax
import jax.numpy as jnp
from jax import lax
from jax.experimental import pallas as pl
from jax.experimental.pallas import tpu as pltpu
from jax.experimental.pallas import tpu_sc as plsc

VOCAB = 100000
BATCH = 16384
DB = 300
DM = 100
DOUT = DB + DM

NC = 2
NS = 16
NW = NC * NS
BPW = BATCH // NW
CH = 128
NCHUNK = BPW // CH
NG = CH // 16

JB = DB // 8 + 1
JM = DM // 8 + 1


def _body(ids_hbm, basic_hbm, modif_hbm, out_hbm,
          ids_v, idxb, idxm, bas_v, mod_v, stage, sem):
    cid = lax.axis_index("c")
    sid = lax.axis_index("s")
    wid = sid * NC + cid
    base = wid * BPW
    pltpu.sync_copy(ids_hbm.at[pl.ds(base, BPW)], ids_v)
    viota = lax.iota(jnp.int32, 16)

    @pl.loop(0, NCHUNK)
    def _chunk(c):
        for g in range(NG):
            col = g * 16 + viota
            w = ids_v[pl.ds(c * CH + g * 16, 16)]
            sb = (w * 75) >> 1
            sm = (w * 25) >> 1
            for j in range(JB):
                plsc.store_scatter(idxb, [jnp.full((16,), j, jnp.int32), col],
                                   sb + j)
            for j in range(JM):
                plsc.store_scatter(idxm, [jnp.full((16,), j, jnp.int32), col],
                                   sm + j)
        cps = [pltpu.async_copy(basic_hbm.at[idxb.at[j]], bas_v.at[j], sem)
               for j in range(JB)]
        cps += [pltpu.async_copy(modif_hbm.at[idxm.at[j]], mod_v.at[j], sem)
                for j in range(JM)]
        for d in cps:
            d.wait()
        for g in range(NG):
            t16 = g * 16 + viota
            w = ids_v[pl.ds(c * CH + g * 16, 16)]
            p = (w & 1) * 4

            @pl.loop(0, DB)
            def _qb(q, p=p, t16=t16):
                fp = p + q
                v = plsc.load_gather(bas_v, [fp >> 3, t16, fp & 7])
                plsc.store_scatter(stage, [t16, jnp.zeros((16,), jnp.int32) + q], v)

            @pl.loop(0, DM)
            def _qm(q, p=p, t16=t16):
                fp = p + q
                v = plsc.load_gather(mod_v, [fp >> 3, t16, fp & 7])
                plsc.store_scatter(stage, [t16, jnp.full((16,), DB, jnp.int32) + q], v)

        pltpu.sync_copy(stage, out_hbm.at[pl.ds(base + c * CH, CH)])


@jax.jit
def kernel(word_ids, basic, modif):
    ids = word_ids.astype(jnp.int32)
    basic8 = basic.reshape(VOCAB * DB // 8, 8)
    modif8 = modif.reshape(VOCAB * DM // 8, 8)
    call = pl.kernel(
        _body,
        out_type=jax.ShapeDtypeStruct((BATCH, DOUT), jnp.float32),
        mesh=plsc.VectorSubcoreMesh(
            core_axis_name="c", subcore_axis_name="s",
            num_cores=NC, num_subcores=NS),
        scratch_types=[
            pltpu.VMEM((BPW,), jnp.int32),
            pltpu.VMEM((JB, CH), jnp.int32),
            pltpu.VMEM((JM, CH), jnp.int32),
            pltpu.VMEM((JB, CH, 8), jnp.float32),
            pltpu.VMEM((JM, CH, 8), jnp.float32),
            pltpu.VMEM((CH, DOUT), jnp.float32),
            pltpu.SemaphoreType.DMA,
        ],
        compiler_params=pltpu.CompilerParams(use_tc_tiling_on_sc=False,
                                             needs_layout_passes=False),
    )
    return call(ids, basic8, modif8)


# speedup vs baseline: 1.0001x; 1.0001x over previous
import jax
import jax.numpy as jnp
from jax import lax
from jax.experimental import pallas as pl
from jax.experimental.pallas import tpu as pltpu
from jax.experimental.pallas import tpu_sc as plsc

VOCAB = 100000
BATCH = 16384
DB = 300
DM = 100
DOUT = DB + DM

NC = 2
NS = 16
NW = NC * NS
BPW = BATCH // NW
CH = 128
NCHUNK = BPW // CH
NG = CH // 16

JB = DB // 8 + 1
JM = DM // 8 + 1


def _body(ids_hbm, basic8, modif8, out_hbm,
          ids_v, idxb, idxm, bas_v, mod_v, stage, sem):
    cid = lax.axis_index("c")
    sid = lax.axis_index("s")
    wid = sid * NC + cid
    base = wid * BPW
    pltpu.sync_copy(ids_hbm.at[pl.ds(base, BPW)], ids_v)
    viota = lax.iota(jnp.int32, 16)

    @pl.loop(0, NCHUNK)
    def _chunk(c):
        for g in range(NG):
            col = g * 16 + viota
            w = ids_v[pl.ds(c * CH + g * 16, 16)]
            sb = (w * 75) >> 1
            sm = (w * 25) >> 1
            for j in range(JB):
                plsc.store_scatter(idxb, [jnp.full((16,), j, jnp.int32), col],
                                   sb + j)
            for j in range(JM):
                plsc.store_scatter(idxm, [jnp.full((16,), j, jnp.int32), col],
                                   sm + j)
        cps = [pltpu.async_copy(basic8.at[idxb.at[j]], bas_v.at[j], sem)
               for j in range(JB)]
        cps += [pltpu.async_copy(modif8.at[idxm.at[j]], mod_v.at[j], sem)
                for j in range(JM)]
        for d in cps:
            d.wait()
        for g in range(NG):
            t16 = g * 16 + viota
            w = ids_v[pl.ds(c * CH + g * 16, 16)]
            p = (w & 1) * 4

            @pl.loop(0, DB)
            def _qb(q, p=p, t16=t16):
                fp = p + q
                v = plsc.load_gather(bas_v, [fp >> 3, t16, fp & 7])
                plsc.store_scatter(stage, [t16, jnp.zeros((16,), jnp.int32) + q], v)

            @pl.loop(0, DM)
            def _qm(q, p=p, t16=t16):
                fp = p + q
                v = plsc.load_gather(mod_v, [fp >> 3, t16, fp & 7])
                plsc.store_scatter(stage, [t16, jnp.full((16,), DB, jnp.int32) + q], v)

        pltpu.sync_copy(stage, out_hbm.at[pl.ds(base + c * CH, CH)])


@jax.jit
def kernel(word_ids, basic, modif):
    ids = word_ids.astype(jnp.int32)
    call = pl.kernel(
        _body,
        out_type=jax.ShapeDtypeStruct((BATCH, DOUT), jnp.float32),
        mesh=plsc.VectorSubcoreMesh(
            core_axis_name="c", subcore_axis_name="s",
            num_cores=NC, num_subcores=NS),
        scratch_types=[
            pltpu.VMEM((BPW,), jnp.int32),
            pltpu.VMEM((JB, CH), jnp.int32),
            pltpu.VMEM((JM, CH), jnp.int32),
            pltpu.VMEM((JB, CH, 8), jnp.float32),
            pltpu.VMEM((JM, CH, 8), jnp.float32),
            pltpu.VMEM((CH, DOUT), jnp.float32),
            pltpu.SemaphoreType.DMA,
        ],
        compiler_params=pltpu.CompilerParams(use_tc_tiling_on_sc=False,
                                             needs_layout_passes=False),
    )
    basic8 = basic.reshape(VOCAB * DB // 8, 8)
    modif8 = modif.reshape(VOCAB * DM // 8, 8)
    return call(ids, basic8, modif8)


# speedup vs baseline: 1.5883x; 1.5880x over previous
import jax
import jax.numpy as jnp
from jax import lax
from jax.experimental import pallas as pl
from jax.experimental.pallas import tpu as pltpu
from jax.experimental.pallas import tpu_sc as plsc

VOCAB = 100000
BATCH = 16384
DB = 300
DM = 100
DOUT = DB + DM

NC = 2
NS = 16
NW = NC * NS
BPW = BATCH // NW
CH = 128
NCHUNK = BPW // CH

VB = 1000


def _concat_body(a_ref, b_ref, o_ref):
    o_ref[...] = jnp.concatenate([a_ref[...], b_ref[...]], axis=1)


def _gather_body(ids_hbm, comb_hbm, out_hbm, ids_v, stage, sem):
    cid = lax.axis_index("c")
    sid = lax.axis_index("s")
    wid = sid * NC + cid
    base = wid * BPW
    pltpu.sync_copy(ids_hbm.at[pl.ds(base, BPW)], ids_v)

    @pl.loop(0, NCHUNK)
    def _chunk(c):
        pltpu.async_copy(
            comb_hbm.at[ids_v.at[pl.ds(c * CH, CH)]], stage, sem).wait()
        pltpu.sync_copy(stage, out_hbm.at[pl.ds(base + c * CH, CH)])


@jax.jit
def kernel(word_ids, basic, modif):
    ids = word_ids.astype(jnp.int32)
    comb = pl.pallas_call(
        _concat_body,
        grid=(VOCAB // VB,),
        in_specs=[
            pl.BlockSpec((VB, DB), lambda i: (i, 0)),
            pl.BlockSpec((VB, DM), lambda i: (i, 0)),
        ],
        out_specs=pl.BlockSpec((VB, DOUT), lambda i: (i, 0)),
        out_shape=jax.ShapeDtypeStruct((VOCAB, DOUT), jnp.float32),
    )(basic, modif)

    call = pl.kernel(
        _gather_body,
        out_type=jax.ShapeDtypeStruct((BATCH, DOUT), jnp.float32),
        mesh=plsc.VectorSubcoreMesh(
            core_axis_name="c", subcore_axis_name="s",
            num_cores=NC, num_subcores=NS),
        scratch_types=[
            pltpu.VMEM((BPW,), jnp.int32),
            pltpu.VMEM((CH, DOUT), jnp.float32),
            pltpu.SemaphoreType.DMA,
        ],
        compiler_params=pltpu.CompilerParams(use_tc_tiling_on_sc=False),
    )
    return call(ids, comb)


# speedup vs baseline: 1.5894x; 1.0007x over previous
import jax
import jax.numpy as jnp
from jax import lax
from jax.experimental import pallas as pl
from jax.experimental.pallas import tpu as pltpu
from jax.experimental.pallas import tpu_sc as plsc

VOCAB = 100000
BATCH = 16384
DB = 300
DM = 100
DOUT = DB + DM

NC = 2
NS = 16
NW = NC * NS
BPW = BATCH // NW
CH = 128
NCHUNK = BPW // CH

VB = 1000


def _concat_body(a_ref, b_ref, o_ref):
    o_ref[...] = jnp.concatenate([a_ref[...], b_ref[...]], axis=1)


def _gather_body(ids_hbm, comb_hbm, out_hbm, ids_v, stage0, stage1,
                 sem0, sem1):
    cid = lax.axis_index("c")
    sid = lax.axis_index("s")
    wid = sid * NC + cid
    base = wid * BPW
    pltpu.sync_copy(ids_hbm.at[pl.ds(base, BPW)], ids_v)

    stages = [stage0, stage1]
    sems = [sem0, sem1]

    def gather(c, b):
        return pltpu.async_copy(
            comb_hbm.at[ids_v.at[pl.ds(c * CH, CH)]], stages[b], sems[b])

    cps = [gather(0, 0), gather(1, 1)]
    for c in range(NCHUNK):
        b = c % 2
        cps[b].wait()
        pltpu.sync_copy(stages[b], out_hbm.at[pl.ds(base + c * CH, CH)])
        if c + 2 < NCHUNK:
            cps[b] = gather(c + 2, b)


@jax.jit
def kernel(word_ids, basic, modif):
    ids = word_ids.astype(jnp.int32)
    comb = pl.pallas_call(
        _concat_body,
        grid=(VOCAB // VB,),
        in_specs=[
            pl.BlockSpec((VB, DB), lambda i: (i, 0)),
            pl.BlockSpec((VB, DM), lambda i: (i, 0)),
        ],
        out_specs=pl.BlockSpec((VB, DOUT), lambda i: (i, 0)),
        out_shape=jax.ShapeDtypeStruct((VOCAB, DOUT), jnp.float32),
    )(basic, modif)

    call = pl.kernel(
        _gather_body,
        out_type=jax.ShapeDtypeStruct((BATCH, DOUT), jnp.float32),
        mesh=plsc.VectorSubcoreMesh(
            core_axis_name="c", subcore_axis_name="s",
            num_cores=NC, num_subcores=NS),
        scratch_types=[
            pltpu.VMEM((BPW,), jnp.int32),
            pltpu.VMEM((CH, DOUT), jnp.float32),
            pltpu.VMEM((CH, DOUT), jnp.float32),
            pltpu.SemaphoreType.DMA,
            pltpu.SemaphoreType.DMA,
        ],
        compiler_params=pltpu.CompilerParams(use_tc_tiling_on_sc=False),
    )
    return call(ids, comb)
